# initial kernel scaffold (unmeasured)
import jax
import jax.numpy as jnp
from jax import lax
from jax.experimental import pallas as pl
from jax.experimental.pallas import tpu as pltpu

N_DEV = 8
M_CHUNK = 512
K = 512
N = 8192


def _body(scale_ref, x_ref, w_ref, out_ref,
          send_buf, recv_buf, send_sems, recv_sems, credit_sem):
    d = lax.axis_index("i")
    left = lax.rem(d + N_DEV - 1, N_DEV)
    right = lax.rem(d + 1, N_DEV)

    barrier = pltpu.get_barrier_semaphore()
    for nbr in (left, right):
        pl.semaphore_signal(barrier, inc=1, device_id=(nbr,),
                            device_id_type=pl.DeviceIdType.MESH)
    pl.semaphore_wait(barrier, 2)

    def partial_chunk(c):
        xc = x_ref[pl.ds(c * M_CHUNK, M_CHUNK), :]
        return lax.dot_general(xc, w_ref[:, :], (((1,), (0,)), ((), ())),
                               preferred_element_type=jnp.float32)

    out_ref[:, :] = partial_chunk(lax.rem(d + N_DEV - 1, N_DEV))

    rdmas = []
    for h in range(N_DEV - 1):
        s = h % 2
        if h >= 2:
            rdmas[h - 2].wait_send()
        send_buf[s, :, :] = out_ref[:, :].astype(jnp.bfloat16)
        if h >= 2:
            pl.semaphore_wait(credit_sem, 1)
        rdma = pltpu.make_async_remote_copy(
            src_ref=send_buf.at[s],
            dst_ref=recv_buf.at[s],
            send_sem=send_sems.at[s],
            recv_sem=recv_sems.at[s],
            device_id=(right,),
            device_id_type=pl.DeviceIdType.MESH,
        )
        rdma.start()
        rdmas.append(rdma)
        p = partial_chunk(lax.rem(d + 2 * N_DEV - h - 2, N_DEV))
        rdma.wait_recv()
        out_ref[:, :] = recv_buf[s, :, :].astype(jnp.float32) + p
        pl.semaphore_signal(credit_sem, inc=1, device_id=(left,),
                            device_id_type=pl.DeviceIdType.MESH)

    rdmas[N_DEV - 3].wait_send()
    rdmas[N_DEV - 2].wait_send()
    pl.semaphore_wait(credit_sem, 2)

    y = out_ref[:, :] * scale_ref[0]
    z = jnp.clip(y, -60.0, 60.0)
    out_ref[:, :] = y / (1.0 + jnp.exp(-z))


def kernel(x, w_mat, scale_x, scale_w):
    scale = (scale_x.reshape(()) * scale_w.reshape(())).reshape(1)
    x_bf = x.astype(jnp.bfloat16)
    w_bf = w_mat.astype(jnp.bfloat16)
    return pl.pallas_call(
        _body,
        out_shape=jax.ShapeDtypeStruct((M_CHUNK, N), jnp.float32),
        in_specs=[
            pl.BlockSpec(memory_space=pltpu.SMEM),
            pl.BlockSpec(memory_space=pltpu.VMEM),
            pl.BlockSpec(memory_space=pltpu.VMEM),
        ],
        out_specs=pl.BlockSpec(memory_space=pltpu.VMEM),
        scratch_shapes=[
            pltpu.VMEM((2, M_CHUNK, N), jnp.bfloat16),
            pltpu.VMEM((2, M_CHUNK, N), jnp.bfloat16),
            pltpu.SemaphoreType.DMA((2,)),
            pltpu.SemaphoreType.DMA((2,)),
            pltpu.SemaphoreType.REGULAR,
        ],
        compiler_params=pltpu.CompilerParams(collective_id=0),
    )(scale, x_bf, w_bf)


# baseline (device time: 728566 ns/iter reference)
import functools

import jax
import jax.numpy as jnp
from jax import lax
from jax.experimental import pallas as pl
from jax.experimental.pallas import tpu as pltpu

N_DEV = 8
M_CHUNK = 512
K = 512
N = 8192
TN = 2048


def _body(scale_ref, x_ref, w_ref, out_ref, comm, send_sems, recv_sems,
          credit_sem):
    d = lax.axis_index("i")
    left = lax.rem(d + N_DEV - 1, N_DEV)
    right = lax.rem(d + 1, N_DEV)

    barrier = pltpu.get_barrier_semaphore()
    for nbr in (left, right):
        pl.semaphore_signal(barrier, inc=1, device_id=(nbr,),
                            device_id_type=pl.DeviceIdType.MESH)
    pl.semaphore_wait(barrier, 2)

    def chunk_rows(c):
        return x_ref[pl.ds(c * M_CHUNK, M_CHUNK), :]

    def dot_tile(xc, t):
        return lax.dot_general(xc, w_ref[:, pl.ds(t, TN)],
                               (((1,), (0,)), ((), ())),
                               preferred_element_type=jnp.float32)

    xc = chunk_rows(lax.rem(d + N_DEV - 1, N_DEV))
    for t in range(0, N, TN):
        out_ref[:, pl.ds(t, TN)] = dot_tile(xc, t)

    for h in range(N_DEV - 1):
        ss = h % 2
        rs = (h + 1) % 2
        for t in range(0, N, TN):
            comm[ss, :, pl.ds(t, TN)] = out_ref[:, pl.ds(t, TN)].astype(
                jnp.bfloat16)
        if h >= 1:
            pl.semaphore_wait(credit_sem, 1)
        rdma = pltpu.make_async_remote_copy(
            src_ref=comm.at[ss],
            dst_ref=comm.at[rs],
            send_sem=send_sems.at[ss],
            recv_sem=recv_sems.at[rs],
            device_id=(right,),
            device_id_type=pl.DeviceIdType.MESH,
        )
        rdma.start()
        rdma.wait_send()
        if h < N_DEV - 2:
            pl.semaphore_signal(credit_sem, inc=1, device_id=(left,),
                                device_id_type=pl.DeviceIdType.MESH)
        rdma.wait_recv()
        xc = chunk_rows(lax.rem(d + 2 * N_DEV - h - 2, N_DEV))
        for t in range(0, N, TN):
            out_ref[:, pl.ds(t, TN)] = (
                comm[rs, :, pl.ds(t, TN)].astype(jnp.float32)
                + dot_tile(xc, t))

    for t in range(0, N, TN):
        y = out_ref[:, pl.ds(t, TN)] * scale_ref[0]
        z = jnp.clip(y, -60.0, 60.0)
        out_ref[:, pl.ds(t, TN)] = y / (1.0 + jnp.exp(-z))

    @functools.partial(pl.run_scoped, exit_sem=pltpu.SemaphoreType.REGULAR)
    def _(exit_sem):
        for nbr in (left, right):
            pl.semaphore_signal(exit_sem, inc=1, device_id=(nbr,),
                                device_id_type=pl.DeviceIdType.MESH)
        pl.semaphore_wait(exit_sem, 2)


def kernel(x, w_mat, scale_x, scale_w):
    scale = (scale_x.reshape(()) * scale_w.reshape(())).reshape(1)
    x_bf = x.astype(jnp.bfloat16)
    w_bf = w_mat.astype(jnp.bfloat16)
    return pl.pallas_call(
        _body,
        out_shape=jax.ShapeDtypeStruct((M_CHUNK, N), jnp.float32),
        in_specs=[
            pl.BlockSpec(memory_space=pltpu.SMEM),
            pl.BlockSpec(memory_space=pltpu.VMEM),
            pl.BlockSpec(memory_space=pltpu.VMEM),
        ],
        out_specs=pl.BlockSpec(memory_space=pltpu.VMEM),
        scratch_shapes=[
            pltpu.VMEM((2, M_CHUNK, N), jnp.bfloat16),
            pltpu.SemaphoreType.DMA((2,)),
            pltpu.SemaphoreType.DMA((2,)),
            pltpu.SemaphoreType.REGULAR,
        ],
        compiler_params=pltpu.CompilerParams(
            collective_id=0, vmem_limit_bytes=100 * 1024 * 1024),
    )(scale, x_bf, w_bf)


# device time: 392678 ns/iter; 1.8554x vs baseline; 1.8554x over previous
import functools

import jax
import jax.numpy as jnp
from jax import lax
from jax.experimental import pallas as pl
from jax.experimental.pallas import tpu as pltpu

N_DEV = 8
M_CHUNK = 512
K = 512
N = 8192
NH = N // 2
TN = 2048


def _body(scale_ref, x_ref, w_ref, out_ref, comm_r, comm_l, p_r, p_l,
          send_sems_r, recv_sems_r, send_sems_l, recv_sems_l,
          credit_r, credit_l):
    d = lax.axis_index("i")
    left = lax.rem(d + N_DEV - 1, N_DEV)
    right = lax.rem(d + 1, N_DEV)

    barrier = pltpu.get_barrier_semaphore()
    for nbr in (left, right):
        pl.semaphore_signal(barrier, inc=1, device_id=(nbr,),
                            device_id_type=pl.DeviceIdType.MESH)
    pl.semaphore_wait(barrier, 2)

    def chunk_rows(c):
        return x_ref[pl.ds(c * M_CHUNK, M_CHUNK), :]

    def dot_tile(xc, col):
        return lax.dot_general(xc, w_ref[:, pl.ds(col, TN)],
                               (((1,), (0,)), ((), ())),
                               preferred_element_type=jnp.float32)

    xc = chunk_rows(lax.rem(d + N_DEV - 1, N_DEV))
    for t in range(0, NH, TN):
        comm_r[0, :, pl.ds(t, TN)] = dot_tile(xc, t).astype(jnp.bfloat16)
    xc = chunk_rows(lax.rem(d + 1, N_DEV))
    for t in range(0, NH, TN):
        comm_l[0, :, pl.ds(t, TN)] = dot_tile(xc, NH + t).astype(jnp.bfloat16)

    for h in range(N_DEV - 1):
        ss = h % 2
        rs = (h + 1) % 2
        if h >= 1:
            pl.semaphore_wait(credit_r, 1)
            pl.semaphore_wait(credit_l, 1)
        rdma_r = pltpu.make_async_remote_copy(
            src_ref=comm_r.at[ss], dst_ref=comm_r.at[rs],
            send_sem=send_sems_r.at[ss], recv_sem=recv_sems_r.at[rs],
            device_id=(right,), device_id_type=pl.DeviceIdType.MESH,
        )
        rdma_l = pltpu.make_async_remote_copy(
            src_ref=comm_l.at[ss], dst_ref=comm_l.at[rs],
            send_sem=send_sems_l.at[ss], recv_sem=recv_sems_l.at[rs],
            device_id=(left,), device_id_type=pl.DeviceIdType.MESH,
        )
        rdma_r.start()
        rdma_l.start()
        xc = chunk_rows(lax.rem(d + 2 * N_DEV - h - 2, N_DEV))
        for t in range(0, NH, TN):
            p_r[:, pl.ds(t, TN)] = dot_tile(xc, t).astype(jnp.bfloat16)
        xc = chunk_rows(lax.rem(d + h + 2, N_DEV))
        for t in range(0, NH, TN):
            p_l[:, pl.ds(t, TN)] = dot_tile(xc, NH + t).astype(jnp.bfloat16)
        rdma_r.wait_send()
        rdma_l.wait_send()
        if h < N_DEV - 2:
            pl.semaphore_signal(credit_r, inc=1, device_id=(left,),
                                device_id_type=pl.DeviceIdType.MESH)
            pl.semaphore_signal(credit_l, inc=1, device_id=(right,),
                                device_id_type=pl.DeviceIdType.MESH)
        rdma_r.wait_recv()
        if h < N_DEV - 2:
            for t in range(0, NH, TN):
                ts = pl.ds(t, TN)
                comm_r[rs, :, ts] = (
                    comm_r[rs, :, ts].astype(jnp.float32)
                    + p_r[:, ts].astype(jnp.float32)).astype(jnp.bfloat16)
        else:
            for t in range(0, NH, TN):
                ts = pl.ds(t, TN)
                out_ref[:, ts] = (comm_r[rs, :, ts].astype(jnp.float32)
                                  + p_r[:, ts].astype(jnp.float32))
        rdma_l.wait_recv()
        if h < N_DEV - 2:
            for t in range(0, NH, TN):
                ts = pl.ds(t, TN)
                comm_l[rs, :, ts] = (
                    comm_l[rs, :, ts].astype(jnp.float32)
                    + p_l[:, ts].astype(jnp.float32)).astype(jnp.bfloat16)
        else:
            for t in range(0, NH, TN):
                ts = pl.ds(t, TN)
                out_ref[:, pl.ds(NH + t, TN)] = (
                    comm_l[rs, :, ts].astype(jnp.float32)
                    + p_l[:, ts].astype(jnp.float32))

    for t in range(0, N, TN):
        ts = pl.ds(t, TN)
        y = out_ref[:, ts] * scale_ref[0]
        z = jnp.clip(y, -60.0, 60.0)
        out_ref[:, ts] = y / (1.0 + jnp.exp(-z))

    @functools.partial(pl.run_scoped, exit_sem=pltpu.SemaphoreType.REGULAR)
    def _(exit_sem):
        for nbr in (left, right):
            pl.semaphore_signal(exit_sem, inc=1, device_id=(nbr,),
                                device_id_type=pl.DeviceIdType.MESH)
        pl.semaphore_wait(exit_sem, 2)


def kernel(x, w_mat, scale_x, scale_w):
    scale = (scale_x.reshape(()) * scale_w.reshape(())).reshape(1)
    x_bf = x.astype(jnp.bfloat16)
    w_bf = w_mat.astype(jnp.bfloat16)
    return pl.pallas_call(
        _body,
        out_shape=jax.ShapeDtypeStruct((M_CHUNK, N), jnp.float32),
        in_specs=[
            pl.BlockSpec(memory_space=pltpu.SMEM),
            pl.BlockSpec(memory_space=pltpu.VMEM),
            pl.BlockSpec(memory_space=pltpu.VMEM),
        ],
        out_specs=pl.BlockSpec(memory_space=pltpu.VMEM),
        scratch_shapes=[
            pltpu.VMEM((2, M_CHUNK, NH), jnp.bfloat16),
            pltpu.VMEM((2, M_CHUNK, NH), jnp.bfloat16),
            pltpu.VMEM((M_CHUNK, NH), jnp.bfloat16),
            pltpu.VMEM((M_CHUNK, NH), jnp.bfloat16),
            pltpu.SemaphoreType.DMA((2,)),
            pltpu.SemaphoreType.DMA((2,)),
            pltpu.SemaphoreType.DMA((2,)),
            pltpu.SemaphoreType.DMA((2,)),
            pltpu.SemaphoreType.REGULAR,
            pltpu.SemaphoreType.REGULAR,
        ],
        compiler_params=pltpu.CompilerParams(
            collective_id=0, vmem_limit_bytes=100 * 1024 * 1024),
    )(scale, x_bf, w_bf)


# device time: 384420 ns/iter; 1.8952x vs baseline; 1.0215x over previous
import functools

import jax
import jax.numpy as jnp
from jax import lax
from jax.experimental import pallas as pl
from jax.experimental.pallas import tpu as pltpu

N_DEV = 8
M_CHUNK = 512
K = 512
N = 8192
NH = N // 2
TN = 2048
N_HOP = N_DEV - 1


def _body(scale_ref, x_ref, w_ref, out_ref, comm_r0, comm_r1, comm_l0,
          comm_l1, p_r, p_l, ss_r0, rs_r0, ss_r1, rs_r1, ss_l0, rs_l0,
          ss_l1, rs_l1, cr_r0, cr_r1, cr_l0, cr_l1):
    d = lax.axis_index("i")
    left = lax.rem(d + N_DEV - 1, N_DEV)
    right = lax.rem(d + 1, N_DEV)

    barrier = pltpu.get_barrier_semaphore()
    for nbr in (left, right):
        pl.semaphore_signal(barrier, inc=1, device_id=(nbr,),
                            device_id_type=pl.DeviceIdType.MESH)
    pl.semaphore_wait(barrier, 2)

    def chunk_rows(c):
        return x_ref[pl.ds(c * M_CHUNK, M_CHUNK), :]

    def dot_tile(xc, col):
        return lax.dot_general(xc, w_ref[:, pl.ds(col, TN)],
                               (((1,), (0,)), ((), ())),
                               preferred_element_type=jnp.float32)

    rings = [
        dict(comm=comm_r0, send_sems=ss_r0, recv_sems=rs_r0, credit=cr_r0,
             fwd=right, bwd=left, p=p_r, pcol=0, col=0),
        dict(comm=comm_l0, send_sems=ss_l0, recv_sems=rs_l0, credit=cr_l0,
             fwd=left, bwd=right, p=p_l, pcol=0, col=NH),
        dict(comm=comm_r1, send_sems=ss_r1, recv_sems=rs_r1, credit=cr_r1,
             fwd=right, bwd=left, p=p_r, pcol=TN, col=TN),
        dict(comm=comm_l1, send_sems=ss_l1, recv_sems=rs_l1, credit=cr_l1,
             fwd=left, bwd=right, p=p_l, pcol=TN, col=NH + TN),
    ]

    def make_rdma(r, h):
        return pltpu.make_async_remote_copy(
            src_ref=r["comm"].at[h % 2],
            dst_ref=r["comm"].at[(h + 1) % 2],
            send_sem=r["send_sems"].at[h % 2],
            recv_sem=r["recv_sems"].at[(h + 1) % 2],
            device_id=(r["fwd"],),
            device_id_type=pl.DeviceIdType.MESH,
        )

    c_r = lax.rem(d + N_DEV - 1, N_DEV)
    c_l = lax.rem(d + 1, N_DEV)
    cur = []
    for r in rings:
        xc = chunk_rows(c_r if r["fwd"] is right else c_l)
        r["comm"][0, :, :] = dot_tile(xc, r["col"]).astype(jnp.bfloat16)
        rdma = make_rdma(r, 0)
        rdma.start()
        cur.append(rdma)

    def prefetch_partials(h):
        xc = chunk_rows(lax.rem(d + 2 * N_DEV - h - 2, N_DEV))
        for t in range(0, NH, TN):
            p_r[:, pl.ds(t, TN)] = dot_tile(xc, t).astype(jnp.bfloat16)
        xc = chunk_rows(lax.rem(d + h + 2, N_DEV))
        for t in range(0, NH, TN):
            p_l[:, pl.ds(t, TN)] = dot_tile(xc, NH + t).astype(jnp.bfloat16)

    prefetch_partials(0)

    for h in range(N_HOP):
        rs = (h + 1) % 2
        if h >= 1:
            for r in rings:
                pl.semaphore_wait(r["credit"], 1)
            cur = [make_rdma(r, h) for r in rings]
            for rdma in cur:
                rdma.start()
            prefetch_partials(h)
        for i, r in enumerate(rings):
            rdma = cur[i]
            rdma.wait_send()
            if h < N_HOP - 1:
                pl.semaphore_signal(r["credit"], inc=1,
                                    device_id=(r["bwd"],),
                                    device_id_type=pl.DeviceIdType.MESH)
        for i, r in enumerate(rings):
            rdma = cur[i]
            rdma.wait_recv()
            ts = pl.ds(r["pcol"], TN)
            if h < N_HOP - 1:
                r["comm"][rs, :, :] = (
                    r["comm"][rs, :, :].astype(jnp.float32)
                    + r["p"][:, ts].astype(jnp.float32)).astype(jnp.bfloat16)
            else:
                v = (r["comm"][rs, :, :].astype(jnp.float32)
                     + r["p"][:, ts].astype(jnp.float32))
                y = v * scale_ref[0]
                z = jnp.clip(y, -60.0, 60.0)
                out_ref[:, pl.ds(r["col"], TN)] = y / (1.0 + jnp.exp(-z))

    @functools.partial(pl.run_scoped, exit_sem=pltpu.SemaphoreType.REGULAR)
    def _(exit_sem):
        for nbr in (left, right):
            pl.semaphore_signal(exit_sem, inc=1, device_id=(nbr,),
                                device_id_type=pl.DeviceIdType.MESH)
        pl.semaphore_wait(exit_sem, 2)


def kernel(x, w_mat, scale_x, scale_w):
    scale = (scale_x.reshape(()) * scale_w.reshape(())).reshape(1)
    x_bf = x.astype(jnp.bfloat16)
    w_bf = w_mat.astype(jnp.bfloat16)
    return pl.pallas_call(
        _body,
        out_shape=jax.ShapeDtypeStruct((M_CHUNK, N), jnp.float32),
        in_specs=[
            pl.BlockSpec(memory_space=pltpu.SMEM),
            pl.BlockSpec(memory_space=pltpu.VMEM),
            pl.BlockSpec(memory_space=pltpu.VMEM),
        ],
        out_specs=pl.BlockSpec(memory_space=pltpu.VMEM),
        scratch_shapes=[
            pltpu.VMEM((2, M_CHUNK, TN), jnp.bfloat16),
            pltpu.VMEM((2, M_CHUNK, TN), jnp.bfloat16),
            pltpu.VMEM((2, M_CHUNK, TN), jnp.bfloat16),
            pltpu.VMEM((2, M_CHUNK, TN), jnp.bfloat16),
            pltpu.VMEM((M_CHUNK, NH), jnp.bfloat16),
            pltpu.VMEM((M_CHUNK, NH), jnp.bfloat16),
            pltpu.SemaphoreType.DMA((2,)), pltpu.SemaphoreType.DMA((2,)),
            pltpu.SemaphoreType.DMA((2,)), pltpu.SemaphoreType.DMA((2,)),
            pltpu.SemaphoreType.DMA((2,)), pltpu.SemaphoreType.DMA((2,)),
            pltpu.SemaphoreType.DMA((2,)), pltpu.SemaphoreType.DMA((2,)),
            pltpu.SemaphoreType.REGULAR, pltpu.SemaphoreType.REGULAR,
            pltpu.SemaphoreType.REGULAR, pltpu.SemaphoreType.REGULAR,
        ],
        compiler_params=pltpu.CompilerParams(
            collective_id=0, vmem_limit_bytes=100 * 1024 * 1024),
    )(scale, x_bf, w_bf)


# device time: 364325 ns/iter; 1.9998x vs baseline; 1.0552x over previous
import functools

import jax
import jax.numpy as jnp
from jax import lax
from jax.experimental import pallas as pl
from jax.experimental.pallas import tpu as pltpu

N_DEV = 8
M_CHUNK = 512
K = 512
N = 8192
NH = N // 2
TN = 2048
N_HOP = N_DEV - 1


def _body(scale_ref, x_ref, w_ref, out_ref, comm_r0, comm_r1, comm_l0,
          comm_l1, p_r, p_l, ss_r0, rs_r0, ss_r1, rs_r1, ss_l0, rs_l0,
          ss_l1, rs_l1, cr_r0, cr_r1, cr_l0, cr_l1):
    d = lax.axis_index("i")
    left = lax.rem(d + N_DEV - 1, N_DEV)
    right = lax.rem(d + 1, N_DEV)

    barrier = pltpu.get_barrier_semaphore()
    for nbr in (left, right):
        pl.semaphore_signal(barrier, inc=1, device_id=(nbr,),
                            device_id_type=pl.DeviceIdType.MESH)
    pl.semaphore_wait(barrier, 2)

    def chunk_rows(c):
        return x_ref[pl.ds(c * M_CHUNK, M_CHUNK), :]

    def dot_tile(xc, col):
        return lax.dot_general(xc, w_ref[:, pl.ds(col, TN)],
                               (((1,), (0,)), ((), ())),
                               preferred_element_type=jnp.float32)

    rings = [
        dict(comm=comm_r0, send_sems=ss_r0, recv_sems=rs_r0, credit=cr_r0,
             fwd=right, bwd=left, p=p_r, pcol=0, col=0),
        dict(comm=comm_l0, send_sems=ss_l0, recv_sems=rs_l0, credit=cr_l0,
             fwd=left, bwd=right, p=p_l, pcol=0, col=NH),
        dict(comm=comm_r1, send_sems=ss_r1, recv_sems=rs_r1, credit=cr_r1,
             fwd=right, bwd=left, p=p_r, pcol=TN, col=TN),
        dict(comm=comm_l1, send_sems=ss_l1, recv_sems=rs_l1, credit=cr_l1,
             fwd=left, bwd=right, p=p_l, pcol=TN, col=NH + TN),
    ]

    def make_rdma(r, h):
        return pltpu.make_async_remote_copy(
            src_ref=r["comm"].at[h % 2],
            dst_ref=r["comm"].at[(h + 1) % 2],
            send_sem=r["send_sems"].at[h % 2],
            recv_sem=r["recv_sems"].at[(h + 1) % 2],
            device_id=(r["fwd"],),
            device_id_type=pl.DeviceIdType.MESH,
        )

    c_r = lax.rem(d + N_DEV - 1, N_DEV)
    c_l = lax.rem(d + 1, N_DEV)
    cur = []
    for r in rings:
        xc = chunk_rows(c_r if r["fwd"] is right else c_l)
        r["comm"][0, :, :] = dot_tile(xc, r["col"]).astype(jnp.bfloat16)
        rdma = make_rdma(r, 0)
        rdma.start()
        cur.append(rdma)

    def prefetch_partials(h):
        xc = chunk_rows(lax.rem(d + 2 * N_DEV - h - 2, N_DEV))
        for t in range(0, NH, TN):
            p_r[:, pl.ds(t, TN)] = dot_tile(xc, t).astype(jnp.bfloat16)
        xc = chunk_rows(lax.rem(d + h + 2, N_DEV))
        for t in range(0, NH, TN):
            p_l[:, pl.ds(t, TN)] = dot_tile(xc, NH + t).astype(jnp.bfloat16)

    prefetch_partials(0)

    for h in range(N_HOP):
        rs = (h + 1) % 2
        nxt = []
        for i, r in enumerate(rings):
            rdma = cur[i]
            rdma.wait_send()
            if h < N_HOP - 1:
                pl.semaphore_signal(r["credit"], inc=1,
                                    device_id=(r["bwd"],),
                                    device_id_type=pl.DeviceIdType.MESH)
            rdma.wait_recv()
            ts = pl.ds(r["pcol"], TN)
            if h < N_HOP - 1:
                r["comm"][rs, :, :] = (
                    r["comm"][rs, :, :].astype(jnp.float32)
                    + r["p"][:, ts].astype(jnp.float32)).astype(jnp.bfloat16)
                pl.semaphore_wait(r["credit"], 1)
                nrdma = make_rdma(r, h + 1)
                nrdma.start()
                nxt.append(nrdma)
            else:
                v = (r["comm"][rs, :, :].astype(jnp.float32)
                     + r["p"][:, ts].astype(jnp.float32))
                y = v * scale_ref[0]
                z = jnp.clip(y, -60.0, 60.0)
                out_ref[:, pl.ds(r["col"], TN)] = y / (1.0 + jnp.exp(-z))
        cur = nxt
        if h < N_HOP - 1:
            prefetch_partials(h + 1)

    @functools.partial(pl.run_scoped, exit_sem=pltpu.SemaphoreType.REGULAR)
    def _(exit_sem):
        for nbr in (left, right):
            pl.semaphore_signal(exit_sem, inc=1, device_id=(nbr,),
                                device_id_type=pl.DeviceIdType.MESH)
        pl.semaphore_wait(exit_sem, 2)


def kernel(x, w_mat, scale_x, scale_w):
    scale = (scale_x.reshape(()) * scale_w.reshape(())).reshape(1)
    x_bf = x.astype(jnp.bfloat16)
    w_bf = w_mat.astype(jnp.bfloat16)
    return pl.pallas_call(
        _body,
        out_shape=jax.ShapeDtypeStruct((M_CHUNK, N), jnp.float32),
        in_specs=[
            pl.BlockSpec(memory_space=pltpu.SMEM),
            pl.BlockSpec(memory_space=pltpu.VMEM),
            pl.BlockSpec(memory_space=pltpu.VMEM),
        ],
        out_specs=pl.BlockSpec(memory_space=pltpu.VMEM),
        scratch_shapes=[
            pltpu.VMEM((2, M_CHUNK, TN), jnp.bfloat16),
            pltpu.VMEM((2, M_CHUNK, TN), jnp.bfloat16),
            pltpu.VMEM((2, M_CHUNK, TN), jnp.bfloat16),
            pltpu.VMEM((2, M_CHUNK, TN), jnp.bfloat16),
            pltpu.VMEM((M_CHUNK, NH), jnp.bfloat16),
            pltpu.VMEM((M_CHUNK, NH), jnp.bfloat16),
            pltpu.SemaphoreType.DMA((2,)), pltpu.SemaphoreType.DMA((2,)),
            pltpu.SemaphoreType.DMA((2,)), pltpu.SemaphoreType.DMA((2,)),
            pltpu.SemaphoreType.DMA((2,)), pltpu.SemaphoreType.DMA((2,)),
            pltpu.SemaphoreType.DMA((2,)), pltpu.SemaphoreType.DMA((2,)),
            pltpu.SemaphoreType.REGULAR, pltpu.SemaphoreType.REGULAR,
            pltpu.SemaphoreType.REGULAR, pltpu.SemaphoreType.REGULAR,
        ],
        compiler_params=pltpu.CompilerParams(
            collective_id=0, vmem_limit_bytes=100 * 1024 * 1024),
    )(scale, x_bf, w_bf)


# device time: 363589 ns/iter; 2.0038x vs baseline; 1.0020x over previous
import functools

import jax
import jax.numpy as jnp
from jax import lax
from jax.experimental import pallas as pl
from jax.experimental.pallas import tpu as pltpu

N_DEV = 8
M_CHUNK = 512
K = 512
N = 8192
NH = N // 2
TN = 2048
N_HOP = N_DEV - 1


def _body(scale_ref, x_ref, w_ref, out_ref, comm_r0, comm_r1, comm_l0,
          comm_l1, p_r, p_l, ss_r0, rs_r0, ss_r1, rs_r1, ss_l0, rs_l0,
          ss_l1, rs_l1, cr_r0, cr_r1, cr_l0, cr_l1):
    d = lax.axis_index("i")

    def f(t):
        return jnp.where(t < 4, t, 11 - t)

    p = f(d)
    right = f(lax.rem(p + 1, N_DEV))
    left = f(lax.rem(p + N_DEV - 1, N_DEV))

    barrier = pltpu.get_barrier_semaphore()
    for nbr in (left, right):
        pl.semaphore_signal(barrier, inc=1, device_id=(nbr,),
                            device_id_type=pl.DeviceIdType.MESH)
    pl.semaphore_wait(barrier, 2)

    def chunk_rows(c):
        return x_ref[pl.ds(c * M_CHUNK, M_CHUNK), :]

    def dot_tile(xc, col):
        return lax.dot_general(xc, w_ref[:, pl.ds(col, TN)],
                               (((1,), (0,)), ((), ())),
                               preferred_element_type=jnp.float32)

    rings = [
        dict(comm=comm_r0, send_sems=ss_r0, recv_sems=rs_r0, credit=cr_r0,
             fwd=right, bwd=left, p=p_r, pcol=0, col=0),
        dict(comm=comm_l0, send_sems=ss_l0, recv_sems=rs_l0, credit=cr_l0,
             fwd=left, bwd=right, p=p_l, pcol=0, col=NH),
        dict(comm=comm_r1, send_sems=ss_r1, recv_sems=rs_r1, credit=cr_r1,
             fwd=right, bwd=left, p=p_r, pcol=TN, col=TN),
        dict(comm=comm_l1, send_sems=ss_l1, recv_sems=rs_l1, credit=cr_l1,
             fwd=left, bwd=right, p=p_l, pcol=TN, col=NH + TN),
    ]

    def make_rdma(r, h):
        return pltpu.make_async_remote_copy(
            src_ref=r["comm"].at[h % 2],
            dst_ref=r["comm"].at[(h + 1) % 2],
            send_sem=r["send_sems"].at[h % 2],
            recv_sem=r["recv_sems"].at[(h + 1) % 2],
            device_id=(r["fwd"],),
            device_id_type=pl.DeviceIdType.MESH,
        )

    c_r = f(lax.rem(p + N_DEV - 1, N_DEV))
    c_l = f(lax.rem(p + 1, N_DEV))
    cur = []
    for r in rings:
        xc = chunk_rows(c_r if r["fwd"] is right else c_l)
        r["comm"][0, :, :] = dot_tile(xc, r["col"]).astype(jnp.bfloat16)
        rdma = make_rdma(r, 0)
        rdma.start()
        cur.append(rdma)

    def prefetch_partials(h):
        xc = chunk_rows(f(lax.rem(p + 2 * N_DEV - h - 2, N_DEV)))
        for t in range(0, NH, TN):
            p_r[:, pl.ds(t, TN)] = dot_tile(xc, t).astype(jnp.bfloat16)
        xc = chunk_rows(f(lax.rem(p + h + 2, N_DEV)))
        for t in range(0, NH, TN):
            p_l[:, pl.ds(t, TN)] = dot_tile(xc, NH + t).astype(jnp.bfloat16)

    prefetch_partials(0)

    for h in range(N_HOP):
        rs = (h + 1) % 2
        nxt = []
        for i, r in enumerate(rings):
            rdma = cur[i]
            rdma.wait_send()
            if h < N_HOP - 1:
                pl.semaphore_signal(r["credit"], inc=1,
                                    device_id=(r["bwd"],),
                                    device_id_type=pl.DeviceIdType.MESH)
            rdma.wait_recv()
            ts = pl.ds(r["pcol"], TN)
            if h < N_HOP - 1:
                r["comm"][rs, :, :] = (
                    r["comm"][rs, :, :].astype(jnp.float32)
                    + r["p"][:, ts].astype(jnp.float32)).astype(jnp.bfloat16)
                pl.semaphore_wait(r["credit"], 1)
                nrdma = make_rdma(r, h + 1)
                nrdma.start()
                nxt.append(nrdma)
            else:
                v = (r["comm"][rs, :, :].astype(jnp.float32)
                     + r["p"][:, ts].astype(jnp.float32))
                y = v * scale_ref[0]
                z = jnp.clip(y, -60.0, 60.0)
                out_ref[:, pl.ds(r["col"], TN)] = y / (1.0 + jnp.exp(-z))
        cur = nxt
        if h < N_HOP - 1:
            prefetch_partials(h + 1)

    @functools.partial(pl.run_scoped, exit_sem=pltpu.SemaphoreType.REGULAR)
    def _(exit_sem):
        for nbr in (left, right):
            pl.semaphore_signal(exit_sem, inc=1, device_id=(nbr,),
                                device_id_type=pl.DeviceIdType.MESH)
        pl.semaphore_wait(exit_sem, 2)


def kernel(x, w_mat, scale_x, scale_w):
    scale = (scale_x.reshape(()) * scale_w.reshape(())).reshape(1)
    x_bf = x.astype(jnp.bfloat16)
    w_bf = w_mat.astype(jnp.bfloat16)
    return pl.pallas_call(
        _body,
        out_shape=jax.ShapeDtypeStruct((M_CHUNK, N), jnp.float32),
        in_specs=[
            pl.BlockSpec(memory_space=pltpu.SMEM),
            pl.BlockSpec(memory_space=pltpu.VMEM),
            pl.BlockSpec(memory_space=pltpu.VMEM),
        ],
        out_specs=pl.BlockSpec(memory_space=pltpu.VMEM),
        scratch_shapes=[
            pltpu.VMEM((2, M_CHUNK, TN), jnp.bfloat16),
            pltpu.VMEM((2, M_CHUNK, TN), jnp.bfloat16),
            pltpu.VMEM((2, M_CHUNK, TN), jnp.bfloat16),
            pltpu.VMEM((2, M_CHUNK, TN), jnp.bfloat16),
            pltpu.VMEM((M_CHUNK, NH), jnp.bfloat16),
            pltpu.VMEM((M_CHUNK, NH), jnp.bfloat16),
            pltpu.SemaphoreType.DMA((2,)), pltpu.SemaphoreType.DMA((2,)),
            pltpu.SemaphoreType.DMA((2,)), pltpu.SemaphoreType.DMA((2,)),
            pltpu.SemaphoreType.DMA((2,)), pltpu.SemaphoreType.DMA((2,)),
            pltpu.SemaphoreType.DMA((2,)), pltpu.SemaphoreType.DMA((2,)),
            pltpu.SemaphoreType.REGULAR, pltpu.SemaphoreType.REGULAR,
            pltpu.SemaphoreType.REGULAR, pltpu.SemaphoreType.REGULAR,
        ],
        compiler_params=pltpu.CompilerParams(
            collective_id=0, vmem_limit_bytes=100 * 1024 * 1024),
    )(scale, x_bf, w_bf)


# device time: 361746 ns/iter; 2.0140x vs baseline; 1.0051x over previous
import functools

import jax
import jax.numpy as jnp
from jax import lax
from jax.experimental import pallas as pl
from jax.experimental.pallas import tpu as pltpu

N_DEV = 8
M_CHUNK = 512
K = 512
N = 8192
NH = N // 2
SUBS = 4
NR = 2 * SUBS
TN = NH // SUBS
N_HOP = N_DEV - 1


def _body(scale_ref, x_ref, w_ref, out_ref, comm, p_r, p_l,
          send_sems, recv_sems, credits):
    d = lax.axis_index("i")

    def f(t):
        return jnp.where(t < 4, t, 11 - t)

    p = f(d)
    right = f(lax.rem(p + 1, N_DEV))
    left = f(lax.rem(p + N_DEV - 1, N_DEV))

    barrier = pltpu.get_barrier_semaphore()
    for nbr in (left, right):
        pl.semaphore_signal(barrier, inc=1, device_id=(nbr,),
                            device_id_type=pl.DeviceIdType.MESH)
    pl.semaphore_wait(barrier, 2)

    def chunk_rows(c):
        return x_ref[pl.ds(c * M_CHUNK, M_CHUNK), :]

    def dot_tile(xc, col):
        return lax.dot_general(xc, w_ref[:, pl.ds(col, TN)],
                               (((1,), (0,)), ((), ())),
                               preferred_element_type=jnp.float32)

    rings = []
    for i in range(NR):
        dir_l = i % 2 == 1
        sub = i // 2
        rings.append(dict(
            idx=i,
            fwd=left if dir_l else right,
            bwd=right if dir_l else left,
            p=p_l if dir_l else p_r,
            pcol=sub * TN,
            col=(NH if dir_l else 0) + sub * TN,
        ))

    def make_rdma(r, h):
        i = r["idx"]
        return pltpu.make_async_remote_copy(
            src_ref=comm.at[i, h % 2],
            dst_ref=comm.at[i, (h + 1) % 2],
            send_sem=send_sems.at[i, h % 2],
            recv_sem=recv_sems.at[i, (h + 1) % 2],
            device_id=(r["fwd"],),
            device_id_type=pl.DeviceIdType.MESH,
        )

    c_r = f(lax.rem(p + N_DEV - 1, N_DEV))
    c_l = f(lax.rem(p + 1, N_DEV))
    cur = []
    xc_r = chunk_rows(c_r)
    xc_l = chunk_rows(c_l)
    for r in rings:
        xc = xc_l if r["fwd"] is left else xc_r
        comm[r["idx"], 0, :, :] = dot_tile(xc, r["col"]).astype(jnp.bfloat16)
        rdma = make_rdma(r, 0)
        rdma.start()
        cur.append(rdma)

    def prefetch_partials(h):
        xc = chunk_rows(f(lax.rem(p + 2 * N_DEV - h - 2, N_DEV)))
        for t in range(0, NH, TN):
            p_r[:, pl.ds(t, TN)] = dot_tile(xc, t).astype(jnp.bfloat16)
        xc = chunk_rows(f(lax.rem(p + h + 2, N_DEV)))
        for t in range(0, NH, TN):
            p_l[:, pl.ds(t, TN)] = dot_tile(xc, NH + t).astype(jnp.bfloat16)

    prefetch_partials(0)

    for h in range(N_HOP):
        rs = (h + 1) % 2
        nxt = []
        for i, r in enumerate(rings):
            rdma = cur[i]
            rdma.wait_send()
            if h < N_HOP - 1:
                pl.semaphore_signal(credits.at[r["idx"]], inc=1,
                                    device_id=(r["bwd"],),
                                    device_id_type=pl.DeviceIdType.MESH)
            rdma.wait_recv()
            ts = pl.ds(r["pcol"], TN)
            if h < N_HOP - 1:
                comm[r["idx"], rs, :, :] = (
                    comm[r["idx"], rs, :, :].astype(jnp.float32)
                    + r["p"][:, ts].astype(jnp.float32)).astype(jnp.bfloat16)
                pl.semaphore_wait(credits.at[r["idx"]], 1)
                nrdma = make_rdma(r, h + 1)
                nrdma.start()
                nxt.append(nrdma)
            else:
                v = (comm[r["idx"], rs, :, :].astype(jnp.float32)
                     + r["p"][:, ts].astype(jnp.float32))
                y = v * scale_ref[0]
                z = jnp.clip(y, -60.0, 60.0)
                out_ref[:, pl.ds(r["col"], TN)] = y / (1.0 + jnp.exp(-z))
        cur = nxt
        if h < N_HOP - 1:
            prefetch_partials(h + 1)

    @functools.partial(pl.run_scoped, exit_sem=pltpu.SemaphoreType.REGULAR)
    def _(exit_sem):
        for nbr in (left, right):
            pl.semaphore_signal(exit_sem, inc=1, device_id=(nbr,),
                                device_id_type=pl.DeviceIdType.MESH)
        pl.semaphore_wait(exit_sem, 2)


def kernel(x, w_mat, scale_x, scale_w):
    scale = (scale_x.reshape(()) * scale_w.reshape(())).reshape(1)
    x_bf = x.astype(jnp.bfloat16)
    w_bf = w_mat.astype(jnp.bfloat16)
    return pl.pallas_call(
        _body,
        out_shape=jax.ShapeDtypeStruct((M_CHUNK, N), jnp.float32),
        in_specs=[
            pl.BlockSpec(memory_space=pltpu.SMEM),
            pl.BlockSpec(memory_space=pltpu.VMEM),
            pl.BlockSpec(memory_space=pltpu.VMEM),
        ],
        out_specs=pl.BlockSpec(memory_space=pltpu.VMEM),
        scratch_shapes=[
            pltpu.VMEM((NR, 2, M_CHUNK, TN), jnp.bfloat16),
            pltpu.VMEM((M_CHUNK, NH), jnp.bfloat16),
            pltpu.VMEM((M_CHUNK, NH), jnp.bfloat16),
            pltpu.SemaphoreType.DMA((NR, 2)),
            pltpu.SemaphoreType.DMA((NR, 2)),
            pltpu.SemaphoreType.REGULAR((NR,)),
        ],
        compiler_params=pltpu.CompilerParams(
            collective_id=0, vmem_limit_bytes=100 * 1024 * 1024),
    )(scale, x_bf, w_bf)
